# baseline (speedup 1.0000x reference)
import functools

import jax
import jax.numpy as jnp
from jax.experimental import pallas as pl
from jax.experimental.pallas import tpu as pltpu

TN = 256


def _matmul_kernel(x_ref, w_ref, o_ref):
    o_ref[...] = jax.lax.dot_general(
        x_ref[...], w_ref[...],
        dimension_numbers=(((1,), (1,)), ((), ())),
        preferred_element_type=jnp.float32,
    )


@jax.jit
def kernel(X, W):
    batch, n_in = X.shape
    n_out = W.shape[0]
    grid = (n_out // TN,)
    return pl.pallas_call(
        _matmul_kernel,
        grid=grid,
        in_specs=[
            pl.BlockSpec((batch, n_in), lambda j: (0, 0)),
            pl.BlockSpec((TN, n_in), lambda j: (j, 0)),
        ],
        out_specs=pl.BlockSpec((batch, TN), lambda j: (0, j)),
        out_shape=jax.ShapeDtypeStruct((batch, n_out), jnp.float32),
        compiler_params=pltpu.CompilerParams(
            dimension_semantics=("arbitrary",),
        ),
    )(X, W)


# speedup vs baseline: 1.1523x; 1.1523x over previous
import functools

import jax
import jax.numpy as jnp
from jax.experimental import pallas as pl
from jax.experimental.pallas import tpu as pltpu

TN = 512


def _matmul_kernel(x_ref, w_ref, o_ref):
    o_ref[...] = jax.lax.dot_general(
        x_ref[...], w_ref[...],
        dimension_numbers=(((1,), (1,)), ((), ())),
        preferred_element_type=jnp.float32,
    )


@jax.jit
def kernel(X, W):
    batch, n_in = X.shape
    n_out = W.shape[0]
    grid = (n_out // TN,)
    return pl.pallas_call(
        _matmul_kernel,
        grid=grid,
        in_specs=[
            pl.BlockSpec((batch, n_in), lambda j: (0, 0)),
            pl.BlockSpec((TN, n_in), lambda j: (j, 0)),
        ],
        out_specs=pl.BlockSpec((batch, TN), lambda j: (0, j)),
        out_shape=jax.ShapeDtypeStruct((batch, n_out), jnp.float32),
        compiler_params=pltpu.CompilerParams(
            dimension_semantics=("arbitrary",),
        ),
    )(X, W)
